# baseline (speedup 1.0000x reference)
import jax
import jax.numpy as jnp
from jax.experimental import pallas as pl
from jax.experimental.pallas import tpu as pltpu

_N_NODES = 10000
_D_FEAT = 128


def _dma_body(x_ref, o_ref, sem):
    cp = pltpu.make_async_copy(x_ref, o_ref, sem)
    cp.start()
    cp.wait()


def kernel(edge_index, r_2, batch, x):
    return pl.pallas_call(
        _dma_body,
        out_shape=jax.ShapeDtypeStruct((_N_NODES, _D_FEAT), jnp.float32),
        in_specs=[pl.BlockSpec(memory_space=pl.ANY)],
        out_specs=pl.BlockSpec(memory_space=pl.ANY),
        scratch_shapes=[pltpu.SemaphoreType.DMA],
    )(x)


# speedup vs baseline: 23.7970x; 23.7970x over previous
import jax
import jax.numpy as jnp
from jax.experimental import pallas as pl

_N_NODES = 10000
_D_FEAT = 128
_BLOCK_ROWS = 2000


def _copy_body(x_ref, o_ref):
    o_ref[...] = x_ref[...]


def kernel(edge_index, r_2, batch, x):
    return pl.pallas_call(
        _copy_body,
        out_shape=jax.ShapeDtypeStruct((_N_NODES, _D_FEAT), jnp.float32),
        grid=(_N_NODES // _BLOCK_ROWS,),
        in_specs=[pl.BlockSpec((_BLOCK_ROWS, _D_FEAT), lambda i: (i, 0))],
        out_specs=pl.BlockSpec((_BLOCK_ROWS, _D_FEAT), lambda i: (i, 0)),
    )(x)


# speedup vs baseline: 36.3927x; 1.5293x over previous
import jax
import jax.numpy as jnp
from jax.experimental import pallas as pl

_N_NODES = 10000
_D_FEAT = 128
_BLOCK_ROWS = 5000


def _copy_body(x_ref, o_ref):
    o_ref[...] = x_ref[...]


def kernel(edge_index, r_2, batch, x):
    return pl.pallas_call(
        _copy_body,
        out_shape=jax.ShapeDtypeStruct((_N_NODES, _D_FEAT), jnp.float32),
        grid=(_N_NODES // _BLOCK_ROWS,),
        in_specs=[pl.BlockSpec((_BLOCK_ROWS, _D_FEAT), lambda i: (i, 0))],
        out_specs=pl.BlockSpec((_BLOCK_ROWS, _D_FEAT), lambda i: (i, 0)),
    )(x)


# speedup vs baseline: 38.7246x; 1.0641x over previous
import jax
import jax.numpy as jnp
from jax.experimental import pallas as pl
from jax.experimental.pallas import tpu as pltpu

_N_NODES = 10000
_D_FEAT = 128
_K = 5
_CH = 2000


def _copy_body(x_ref, o_ref, buf, in_sem, out_sem):
    for i in range(_K):
        pltpu.make_async_copy(
            x_ref.at[pl.ds(i * _CH, _CH), :], buf.at[i], in_sem.at[i]
        ).start()
    for i in range(_K):
        pltpu.make_async_copy(
            x_ref.at[pl.ds(i * _CH, _CH), :], buf.at[i], in_sem.at[i]
        ).wait()
        pltpu.make_async_copy(
            buf.at[i], o_ref.at[pl.ds(i * _CH, _CH), :], out_sem.at[i]
        ).start()
    for i in range(_K):
        pltpu.make_async_copy(
            buf.at[i], o_ref.at[pl.ds(i * _CH, _CH), :], out_sem.at[i]
        ).wait()


def kernel(edge_index, r_2, batch, x):
    return pl.pallas_call(
        _copy_body,
        out_shape=jax.ShapeDtypeStruct((_N_NODES, _D_FEAT), jnp.float32),
        in_specs=[pl.BlockSpec(memory_space=pl.ANY)],
        out_specs=pl.BlockSpec(memory_space=pl.ANY),
        scratch_shapes=[
            pltpu.MemorySpace.VMEM((_K, _CH, _D_FEAT), jnp.float32),
            pltpu.SemaphoreType.DMA((_K,)),
            pltpu.SemaphoreType.DMA((_K,)),
        ],
    )(x)
